# baseline (speedup 1.0000x reference)
import functools
import math

import jax
import jax.numpy as jnp
from jax import lax
from jax.experimental import pallas as pl
from jax.experimental.pallas import tpu as pltpu
from jax.experimental.pallas import tpu_sc as plsc

N = 4096
E = 65536
D = 768
H = 4
L = 16
B = 8
M = 16
DH = D // H

BN = 64
BNB = 512
KB = 768


def _mha_body(x_ref, nb_ref, c_ref, wq_ref, bq_ref, wk_ref, bk_ref,
              wv_ref, bv_ref, wo_ref, bo_ref, out_ref):
    x3 = x_ref[...]
    c3 = c_ref[...]
    nb = nb_ref[...]
    onehot = (nb == jax.lax.broadcasted_iota(jnp.int32, (1, B), 1)
              ).astype(jnp.bfloat16)
    scale = 1.0 / math.sqrt(DH)
    acc = None
    for h in range(H):
        wq_h = wq_ref[...][h]
        wk_h = wk_ref[...][h]
        wv_h = wv_ref[...][h]
        woT_h = wo_ref[...][h]
        bq_h = bq_ref[...][h][None]
        bk_h = bk_ref[...][h][None]
        bv_h = bv_ref[...][h][None]
        q_h = jax.lax.dot_general(
            x3, wq_h, (((2,), (1,)), ((), ())),
            preferred_element_type=jnp.float32) + bq_h[None]
        ck_h = jax.lax.dot_general(
            c3, wk_h, (((2,), (1,)), ((), ())),
            preferred_element_type=jnp.float32) + bk_h[None]
        cv_h = jax.lax.dot_general(
            c3, wv_h, (((2,), (1,)), ((), ())),
            preferred_element_type=jnp.float32) + bv_h[None]
        kb_h = jax.lax.dot_general(
            onehot, ck_h.astype(jnp.bfloat16), (((1,), (0,)), ((), ())),
            preferred_element_type=jnp.float32)
        vb_h = jax.lax.dot_general(
            onehot, cv_h.astype(jnp.bfloat16), (((1,), (0,)), ((), ())),
            preferred_element_type=jnp.float32)
        s = jax.lax.dot_general(
            q_h.astype(jnp.bfloat16), kb_h.astype(jnp.bfloat16),
            (((2,), (2,)), ((0,), (0,))),
            preferred_element_type=jnp.float32) * scale
        s = s - jnp.max(s, axis=-1, keepdims=True)
        e = jnp.exp(s)
        attn = e / jnp.sum(e, axis=-1, keepdims=True)
        o_h = jax.lax.dot_general(
            attn.astype(jnp.bfloat16), vb_h.astype(jnp.bfloat16),
            (((2,), (1,)), ((0,), (0,))),
            preferred_element_type=jnp.float32)
        part = jax.lax.dot_general(
            o_h.astype(jnp.bfloat16), woT_h, (((2,), (0,)), ((), ())),
            preferred_element_type=jnp.float32)
        acc = part if acc is None else acc + part
    out_ref[...] = (acc + bo_ref[...][None]).astype(jnp.bfloat16)


def _mha_call(x3, nb2, c, wq3, bq2, wk3, bk2, wv3, bv2, woT3, bo2):
    grid = (N // BN,)
    return pl.pallas_call(
        _mha_body,
        grid=grid,
        in_specs=[
            pl.BlockSpec((BN, L, D), lambda i: (i, 0, 0)),
            pl.BlockSpec((BN, 1), lambda i: (i, 0)),
            pl.BlockSpec((B, M, D), lambda i: (0, 0, 0)),
            pl.BlockSpec((H, DH, D), lambda i: (0, 0, 0)),
            pl.BlockSpec((H, DH), lambda i: (0, 0)),
            pl.BlockSpec((H, DH, D), lambda i: (0, 0, 0)),
            pl.BlockSpec((H, DH), lambda i: (0, 0)),
            pl.BlockSpec((H, DH, D), lambda i: (0, 0, 0)),
            pl.BlockSpec((H, DH), lambda i: (0, 0)),
            pl.BlockSpec((H, DH, D), lambda i: (0, 0, 0)),
            pl.BlockSpec((1, D), lambda i: (0, 0)),
        ],
        out_specs=pl.BlockSpec((BN, L, D), lambda i: (i, 0, 0)),
        out_shape=jax.ShapeDtypeStruct((N, L, D), jnp.bfloat16),
    )(x3, nb2, c, wq3, bq2, wk3, bk2, wv3, bv2, woT3, bo2)


def _gat_gemm_body(x_ref, w_ref, asrc_ref, adst_ref,
                   h_ref, a_src_ref, a_dst_ref, acc_ref):
    kb = pl.program_id(2)

    @pl.when(kb == 0)
    def _():
        acc_ref[...] = jnp.zeros_like(acc_ref)

    acc_ref[...] += jax.lax.dot_general(
        x_ref[...], w_ref[...], (((1,), (1,)), ((), ())),
        preferred_element_type=jnp.float32)

    @pl.when(kb == (L * D // KB) - 1)
    def _():
        h_blk = acc_ref[...]
        h_ref[...] = h_blk[None]
        a_src_ref[...] = jax.lax.dot_general(
            asrc_ref[...][0], h_blk, (((1,), (1,)), ((), ())),
            preferred_element_type=jnp.float32)[None]
        a_dst_ref[...] = jax.lax.dot_general(
            adst_ref[...][0], h_blk, (((1,), (1,)), ((), ())),
            preferred_element_type=jnp.float32)[None]


def _gat_gemm_call(att_x2, w_gat, asrc2, adst2):
    grid = (H, N // BNB, L * D // KB)
    return pl.pallas_call(
        _gat_gemm_body,
        grid=grid,
        in_specs=[
            pl.BlockSpec((BNB, KB), lambda h, nb, kb: (nb, kb)),
            pl.BlockSpec((D, KB), lambda h, nb, kb: (h, kb)),
            pl.BlockSpec((1, 1, D), lambda h, nb, kb: (h, 0, 0)),
            pl.BlockSpec((1, 1, D), lambda h, nb, kb: (h, 0, 0)),
        ],
        out_specs=[
            pl.BlockSpec((1, BNB, D), lambda h, nb, kb: (h, nb, 0)),
            pl.BlockSpec((1, 1, BNB), lambda h, nb, kb: (h, 0, nb)),
            pl.BlockSpec((1, 1, BNB), lambda h, nb, kb: (h, 0, nb)),
        ],
        out_shape=[
            jax.ShapeDtypeStruct((H, N, D), jnp.float32),
            jax.ShapeDtypeStruct((H, 1, N), jnp.float32),
            jax.ShapeDtypeStruct((H, 1, N), jnp.float32),
        ],
        scratch_shapes=[pltpu.VMEM((BNB, D), jnp.float32)],
    )(att_x2, w_gat, asrc2, adst2)


E2 = E + N
NT = 16
EPT = E2 // NT
GK = 16
NG = EPT // GK
NHALF = N // 2
RPT = NHALF // NT
HN = H * N


def _sc_coef_body(src_hbm, dst_hbm, asrc_hbm, adst_hbm, cmax_hbm,
                  coef_hbm,
                  src_ref, dst_ref, asrc_ref, adst_ref, denom_ref, tmp_ref,
                  expa_ref, cmax_ref, shared_den):
    sid = lax.axis_index("s")

    pltpu.sync_copy(src_hbm.at[pl.ds(sid * EPT, EPT)], src_ref)
    pltpu.sync_copy(dst_hbm.at[pl.ds(sid * EPT, EPT)], dst_ref)
    pltpu.sync_copy(asrc_hbm, asrc_ref)
    pltpu.sync_copy(adst_hbm, adst_ref)
    pltpu.sync_copy(cmax_hbm, cmax_ref)

    zeros = jnp.zeros((16,), jnp.float32)

    def _zero_denom(i, _):
        denom_ref[pl.ds(i * 16, 16)] = zeros
        return 0
    lax.fori_loop(0, HN // 16, _zero_denom, 0)

    def _p1(g, _):
        off = g * GK
        src_v = src_ref[pl.ds(off, 16)]
        dst_v = dst_ref[pl.ds(off, 16)]
        for h in range(H):
            av = plsc.load_gather(asrc_ref, [src_v + h * N])
            bv = plsc.load_gather(adst_ref, [dst_v + h * N])
            sv = av + bv
            alpha = jnp.where(sv >= 0.0, sv, sv * jnp.float32(0.2))
            ex = jnp.exp(alpha - cmax_ref[...][h])
            expa_ref[pl.ds(h * EPT + off, 16)] = ex
            plsc.addupdate_scatter(denom_ref, [dst_v + h * N], ex)
        return 0
    lax.fori_loop(0, NG, _p1, 0)

    pltpu.sync_copy(denom_ref, shared_den.at[sid])
    plsc.subcore_barrier()

    def _zero_denom2(i, _):
        denom_ref[pl.ds(i * 16, 16)] = zeros
        return 0
    lax.fori_loop(0, HN // 16, _zero_denom2, 0)

    def _accum_tile(t, _):
        pltpu.sync_copy(shared_den.at[t], tmp_ref)

        def _acc(i, _2):
            sl = pl.ds(i * 16, 16)
            denom_ref[sl] = denom_ref[sl] + tmp_ref[sl]
            return 0
        lax.fori_loop(0, HN // 16, _acc, 0)
        return 0
    lax.fori_loop(0, NT, _accum_tile, 0)

    def _csweep(g, _):
        off = g * GK
        dst_v = dst_ref[pl.ds(off, 16)]
        for h in range(H):
            sl = pl.ds(h * EPT + off, 16)
            den = plsc.load_gather(denom_ref, [dst_v + h * N])
            expa_ref[sl] = expa_ref[sl] / (den + jnp.float32(1e-16))
        return 0
    lax.fori_loop(0, NG, _csweep, 0)

    for h in range(H):
        pltpu.sync_copy(expa_ref.at[pl.ds(h * EPT, EPT)],
                        coef_hbm.at[pl.ds(h * E2 + sid * EPT, EPT)])


NW = 2 * NT
RPW = N // NW
CHK = 2048
NCHK = E2 // CHK


def _sc_agg_body(src_hbm, dst_hbm, coef_hbm, hrows_hbm,
                 out_hbm,
                 srcc_ref, dstc_ref, coefc_ref, rows0_ref, rows1_ref,
                 idx0_ref, idx1_ref, pd0_ref, pd1_ref, pc0_ref, pc1_ref,
                 ps_ref, pd_ref, pc_ref, acc_ref, sem0, sem1, semc):
    cid = lax.axis_index("c")
    sid = lax.axis_index("s")
    w = cid * NT + sid
    base = w * RPW
    zeros = jnp.zeros((16,), jnp.float32)
    slots = ((rows0_ref, idx0_ref, pd0_ref, pc0_ref, sem0),
             (rows1_ref, idx1_ref, pd1_ref, pc1_ref, sem1))

    def _start(slot, h):
        rows, idx, pds, pcs, sem = slots[slot]
        idx[...] = ps_ref[pl.ds(0, 16)] + h * N
        pds[...] = pd_ref[pl.ds(0, 16)]
        pcs[...] = pc_ref[pl.ds(0, 16)]
        pltpu.async_copy(hrows_hbm.at[idx], rows, sem)

    def _acc(slot):
        rows, idx, pds, pcs, sem = slots[slot]
        pltpu.make_async_copy(hrows_hbm.at[idx], rows, sem).wait()

        def _row(k, _):
            dl = pds[pl.ds(k, 16)][0]
            ck = pcs[pl.ds(k, 16)][0]
            ab = dl * D
            for j in range(D // 16):
                sl = pl.ds(ab + j * 16, 16)
                acc_ref[sl] = (acc_ref[sl]
                               + rows[k, pl.ds(j * 16, 16)] * ck)
            return 0
        lax.fori_loop(0, GK, _row, 0)

    def _fire(par, nout, h):
        @pl.when((nout == 2) & (par == 0))
        def _():
            _acc(0)

        @pl.when((nout == 2) & (par == 1))
        def _():
            _acc(1)

        @pl.when(par == 0)
        def _():
            _start(0, h)

        @pl.when(par == 1)
        def _():
            _start(1, h)

    def _head(h, _hc):
        def _zero(i, _):
            for j in range(8):
                acc_ref[pl.ds(i * 128 + j * 16, 16)] = zeros
            return 0
        lax.fori_loop(0, RPW * D // 128, _zero, 0)

        def _chunk(cki, carry):
            c1 = pltpu.async_copy(src_hbm.at[pl.ds(cki * CHK, CHK)],
                                  srcc_ref, semc)
            c2 = pltpu.async_copy(dst_hbm.at[pl.ds(cki * CHK, CHK)],
                                  dstc_ref, semc)
            c3 = pltpu.async_copy(coef_hbm.at[pl.ds(h * E2 + cki * CHK, CHK)],
                                  coefc_ref, semc)
            c1.wait()
            c2.wait()
            c3.wait()

            def _grp(i, carry2):
                carry3 = carry2
                for u in range(2):
                    np2, par, nout = carry3
                    off = i * 32 + u * 16
                    dst_v = dstc_ref[pl.ds(off, 16)]
                    dloc = dst_v - base
                    inr = (dloc >= 0) & (dloc < RPW)
                    cnt = plsc.all_reduce_population_count(inr)[0]
                    plsc.store_compressed(ps_ref.at[pl.ds(np2, 16)],
                                          srcc_ref[pl.ds(off, 16)], mask=inr)
                    plsc.store_compressed(pd_ref.at[pl.ds(np2, 16)], dloc,
                                          mask=inr)
                    plsc.store_compressed(pc_ref.at[pl.ds(np2, 16)],
                                          coefc_ref[pl.ds(off, 16)], mask=inr)
                    np3 = np2 + cnt
                    fired = np3 >= GK

                    @pl.when(fired)
                    def _():
                        _fire(par, nout, h)
                        ps_ref[pl.ds(0, 16)] = ps_ref[pl.ds(16, 16)]
                        pd_ref[pl.ds(0, 16)] = pd_ref[pl.ds(16, 16)]
                        pc_ref[pl.ds(0, 16)] = pc_ref[pl.ds(16, 16)]
                    carry3 = (jnp.where(fired, np3 - GK, np3),
                              jnp.where(fired, 1 - par, par),
                              jnp.where(fired,
                                        jnp.minimum(nout + 1, 2), nout))
                return carry3
            return lax.fori_loop(0, CHK // 32, _grp, carry)
        np_f, par_f, nout_f = lax.fori_loop(
            0, NCHK, _chunk,
            (jnp.int32(0), jnp.int32(0), jnp.int32(0)))

        @pl.when(np_f > 0)
        def _():
            lanes = lax.iota(jnp.int32, 16)
            valid = lanes < np_f
            pc_ref[pl.ds(0, 16)] = jnp.where(valid, pc_ref[pl.ds(0, 16)],
                                             jnp.float32(0.0))
            ps_ref[pl.ds(0, 16)] = jnp.where(valid, ps_ref[pl.ds(0, 16)], 0)
            pd_ref[pl.ds(0, 16)] = jnp.where(valid, pd_ref[pl.ds(0, 16)], 0)
            _fire(par_f, nout_f, h)

        fired_t = np_f > 0
        par_t = jnp.where(fired_t, 1 - par_f, par_f)
        nout_t = jnp.where(fired_t, jnp.minimum(nout_f + 1, 2), nout_f)

        @pl.when((nout_t == 2) & (par_t == 0))
        def _():
            _acc(0)

        @pl.when((nout_t == 2) & (par_t == 1))
        def _():
            _acc(1)

        @pl.when((nout_t >= 1) & (par_t == 0))
        def _():
            _acc(1)

        @pl.when((nout_t >= 1) & (par_t == 1))
        def _():
            _acc(0)

        pltpu.sync_copy(acc_ref,
                        out_hbm.at[pl.ds((h * N + base) * D, RPW * D)])
        return 0
    lax.fori_loop(0, H, _head, 0)


def _sc_edge_call(src, dst, asrc_flat, adst_flat, cmax16, hrows):
    mesh = plsc.VectorSubcoreMesh(core_axis_name="c", subcore_axis_name="s")
    coef = functools.partial(
        pl.kernel,
        out_type=jax.ShapeDtypeStruct((H * E2,), jnp.float32),
        mesh=mesh,
        compiler_params=pltpu.CompilerParams(needs_layout_passes=False),
        scratch_types=[
            pltpu.VMEM((EPT,), jnp.int32),
            pltpu.VMEM((EPT,), jnp.int32),
            pltpu.VMEM((HN,), jnp.float32),
            pltpu.VMEM((HN,), jnp.float32),
            pltpu.VMEM((HN,), jnp.float32),
            pltpu.VMEM((HN,), jnp.float32),
            pltpu.VMEM((H * EPT,), jnp.float32),
            pltpu.VMEM((16,), jnp.float32),
            pltpu.VMEM_SHARED((NT, HN), jnp.float32),
        ],
    )(_sc_coef_body)(src, dst, asrc_flat, adst_flat, cmax16)

    agg = functools.partial(
        pl.kernel,
        out_type=jax.ShapeDtypeStruct((HN * D,), jnp.float32),
        mesh=mesh,
        compiler_params=pltpu.CompilerParams(needs_layout_passes=False),
        scratch_types=[
            pltpu.VMEM((CHK,), jnp.int32),
            pltpu.VMEM((CHK,), jnp.int32),
            pltpu.VMEM((CHK,), jnp.float32),
            pltpu.VMEM((GK, D), jnp.float32),
            pltpu.VMEM((GK, D), jnp.float32),
            pltpu.VMEM((GK,), jnp.int32),
            pltpu.VMEM((GK,), jnp.int32),
            pltpu.VMEM((GK,), jnp.int32),
            pltpu.VMEM((GK,), jnp.int32),
            pltpu.VMEM((GK,), jnp.float32),
            pltpu.VMEM((GK,), jnp.float32),
            pltpu.VMEM((2 * GK,), jnp.int32),
            pltpu.VMEM((2 * GK,), jnp.int32),
            pltpu.VMEM((2 * GK,), jnp.float32),
            pltpu.VMEM((RPW * D,), jnp.float32),
            pltpu.SemaphoreType.DMA,
            pltpu.SemaphoreType.DMA,
            pltpu.SemaphoreType.DMA,
        ],
    )(_sc_agg_body)(src, dst, coef, hrows)
    return agg


def kernel(x, edge_index, edge_attr, c, node_batch, Wq, bq, Wk, bk, Wv, bv,
           Wo, bo, W_gat, att_src, att_dst, b_gat):
    del edge_attr
    x3 = x.reshape(N, L, D).astype(jnp.bfloat16)
    nb2 = node_batch.reshape(N, 1)
    wq3 = Wq.reshape(H, DH, D).astype(jnp.bfloat16)
    wk3 = Wk.reshape(H, DH, D).astype(jnp.bfloat16)
    wv3 = Wv.reshape(H, DH, D).astype(jnp.bfloat16)
    woT3 = Wo.T.reshape(H, DH, D).astype(jnp.bfloat16)
    bq2 = bq.reshape(H, DH)
    bk2 = bk.reshape(H, DH)
    bv2 = bv.reshape(H, DH)
    bo2 = bo.reshape(1, D)

    c16 = c.astype(jnp.bfloat16)
    att_x = _mha_call(x3, nb2, c16, wq3, bq2, wk3, bk2, wv3, bv2, woT3, bo2)
    att_x2 = att_x.reshape(N, L * D)

    asrc2 = att_src.reshape(H, 1, D)
    adst2 = att_dst.reshape(H, 1, D)
    h_out, a_srcT, a_dstT = _gat_gemm_call(
        att_x2, W_gat.astype(jnp.bfloat16), asrc2, adst2)

    a_src_flat = a_srcT.reshape(H * N)
    a_dst_flat = a_dstT.reshape(H * N)
    cmax = jax.nn.relu(jnp.max(a_srcT.reshape(H, N), axis=1) +
                       jnp.max(a_dstT.reshape(H, N), axis=1))
    cmax16 = jnp.concatenate([cmax, jnp.zeros((16 - H,), jnp.float32)])
    loop = jnp.arange(N, dtype=edge_index.dtype)
    src = jnp.concatenate([edge_index[0], loop])
    dst = jnp.concatenate([edge_index[1], loop])
    hrows = h_out.reshape(H * N, D)
    agg = _sc_edge_call(src, dst, a_src_flat, a_dst_flat, cmax16, hrows)
    out = agg.reshape(H, N, D).transpose(1, 0, 2).reshape(N, H * D)
    return out + b_gat


# speedup vs baseline: 1.0581x; 1.0581x over previous
import functools
import math

import jax
import jax.numpy as jnp
from jax import lax
from jax.experimental import pallas as pl
from jax.experimental.pallas import tpu as pltpu
from jax.experimental.pallas import tpu_sc as plsc

N = 4096
E = 65536
D = 768
H = 4
L = 16
B = 8
M = 16
DH = D // H

BN = 128
BNB = 1024
KB = 768


def _mha_body(x_ref, nb_ref, c_ref, wq_ref, bq_ref, wk_ref, bk_ref,
              wv_ref, bv_ref, wo_ref, bo_ref, out_ref):
    x3 = x_ref[...]
    c3 = c_ref[...]
    nb = nb_ref[...]
    onehot = (nb == jax.lax.broadcasted_iota(jnp.int32, (1, B), 1)
              ).astype(jnp.bfloat16)
    scale = 1.0 / math.sqrt(DH)
    acc = None
    for h in range(H):
        wq_h = wq_ref[...][h]
        wk_h = wk_ref[...][h]
        wv_h = wv_ref[...][h]
        woT_h = wo_ref[...][h]
        bq_h = bq_ref[...][h][None]
        bk_h = bk_ref[...][h][None]
        bv_h = bv_ref[...][h][None]
        q_h = jax.lax.dot_general(
            x3, wq_h, (((2,), (1,)), ((), ())),
            preferred_element_type=jnp.float32) + bq_h[None]
        ck_h = jax.lax.dot_general(
            c3, wk_h, (((2,), (1,)), ((), ())),
            preferred_element_type=jnp.float32) + bk_h[None]
        cv_h = jax.lax.dot_general(
            c3, wv_h, (((2,), (1,)), ((), ())),
            preferred_element_type=jnp.float32) + bv_h[None]
        kb_h = jax.lax.dot_general(
            onehot, ck_h.astype(jnp.bfloat16), (((1,), (0,)), ((), ())),
            preferred_element_type=jnp.float32)
        vb_h = jax.lax.dot_general(
            onehot, cv_h.astype(jnp.bfloat16), (((1,), (0,)), ((), ())),
            preferred_element_type=jnp.float32)
        s = jax.lax.dot_general(
            q_h.astype(jnp.bfloat16), kb_h.astype(jnp.bfloat16),
            (((2,), (2,)), ((0,), (0,))),
            preferred_element_type=jnp.float32) * scale
        s = s - jnp.max(s, axis=-1, keepdims=True)
        e = jnp.exp(s)
        attn = e / jnp.sum(e, axis=-1, keepdims=True)
        o_h = jax.lax.dot_general(
            attn.astype(jnp.bfloat16), vb_h.astype(jnp.bfloat16),
            (((2,), (1,)), ((0,), (0,))),
            preferred_element_type=jnp.float32)
        part = jax.lax.dot_general(
            o_h.astype(jnp.bfloat16), woT_h, (((2,), (0,)), ((), ())),
            preferred_element_type=jnp.float32)
        acc = part if acc is None else acc + part
    out_ref[...] = (acc + bo_ref[...][None]).astype(jnp.bfloat16)


def _mha_call(x3, nb2, c, wq3, bq2, wk3, bk2, wv3, bv2, woT3, bo2):
    grid = (N // BN,)
    return pl.pallas_call(
        _mha_body,
        grid=grid,
        in_specs=[
            pl.BlockSpec((BN, L, D), lambda i: (i, 0, 0)),
            pl.BlockSpec((BN, 1), lambda i: (i, 0)),
            pl.BlockSpec((B, M, D), lambda i: (0, 0, 0)),
            pl.BlockSpec((H, DH, D), lambda i: (0, 0, 0)),
            pl.BlockSpec((H, DH), lambda i: (0, 0)),
            pl.BlockSpec((H, DH, D), lambda i: (0, 0, 0)),
            pl.BlockSpec((H, DH), lambda i: (0, 0)),
            pl.BlockSpec((H, DH, D), lambda i: (0, 0, 0)),
            pl.BlockSpec((H, DH), lambda i: (0, 0)),
            pl.BlockSpec((H, DH, D), lambda i: (0, 0, 0)),
            pl.BlockSpec((1, D), lambda i: (0, 0)),
        ],
        out_specs=pl.BlockSpec((BN, L, D), lambda i: (i, 0, 0)),
        out_shape=jax.ShapeDtypeStruct((N, L, D), jnp.bfloat16),
    )(x3, nb2, c, wq3, bq2, wk3, bk2, wv3, bv2, woT3, bo2)


def _gat_gemm_body(x_ref, w_ref, asrc_ref, adst_ref,
                   h_ref, a_src_ref, a_dst_ref, acc_ref):
    kb = pl.program_id(2)

    @pl.when(kb == 0)
    def _():
        acc_ref[...] = jnp.zeros_like(acc_ref)

    acc_ref[...] += jax.lax.dot_general(
        x_ref[...], w_ref[...], (((1,), (1,)), ((), ())),
        preferred_element_type=jnp.float32)

    @pl.when(kb == (L * D // KB) - 1)
    def _():
        h_blk = acc_ref[...]
        h_ref[...] = h_blk[None]
        a_src_ref[...] = jax.lax.dot_general(
            asrc_ref[...][0], h_blk, (((1,), (1,)), ((), ())),
            preferred_element_type=jnp.float32)[None]
        a_dst_ref[...] = jax.lax.dot_general(
            adst_ref[...][0], h_blk, (((1,), (1,)), ((), ())),
            preferred_element_type=jnp.float32)[None]


def _gat_gemm_call(att_x2, w_gat, asrc2, adst2):
    grid = (H, N // BNB, L * D // KB)
    return pl.pallas_call(
        _gat_gemm_body,
        grid=grid,
        in_specs=[
            pl.BlockSpec((BNB, KB), lambda h, nb, kb: (nb, kb)),
            pl.BlockSpec((D, KB), lambda h, nb, kb: (h, kb)),
            pl.BlockSpec((1, 1, D), lambda h, nb, kb: (h, 0, 0)),
            pl.BlockSpec((1, 1, D), lambda h, nb, kb: (h, 0, 0)),
        ],
        out_specs=[
            pl.BlockSpec((1, BNB, D), lambda h, nb, kb: (h, nb, 0)),
            pl.BlockSpec((1, 1, BNB), lambda h, nb, kb: (h, 0, nb)),
            pl.BlockSpec((1, 1, BNB), lambda h, nb, kb: (h, 0, nb)),
        ],
        out_shape=[
            jax.ShapeDtypeStruct((H, N, D), jnp.float32),
            jax.ShapeDtypeStruct((H, 1, N), jnp.float32),
            jax.ShapeDtypeStruct((H, 1, N), jnp.float32),
        ],
        scratch_shapes=[pltpu.VMEM((BNB, D), jnp.float32)],
    )(att_x2, w_gat, asrc2, adst2)


E2 = E + N
NT = 16
EPT = E2 // NT
GK = 16
NG = EPT // GK
NHALF = N // 2
RPT = NHALF // NT
HN = H * N


def _sc_coef_body(src_hbm, dst_hbm, asrc_hbm, adst_hbm, cmax_hbm,
                  coef_hbm,
                  src_ref, dst_ref, asrc_ref, adst_ref, denom_ref, tmp_ref,
                  expa_ref, cmax_ref, shared_den):
    sid = lax.axis_index("s")

    pltpu.sync_copy(src_hbm.at[pl.ds(sid * EPT, EPT)], src_ref)
    pltpu.sync_copy(dst_hbm.at[pl.ds(sid * EPT, EPT)], dst_ref)
    pltpu.sync_copy(asrc_hbm, asrc_ref)
    pltpu.sync_copy(adst_hbm, adst_ref)
    pltpu.sync_copy(cmax_hbm, cmax_ref)

    zeros = jnp.zeros((16,), jnp.float32)

    def _zero_denom(i, _):
        denom_ref[pl.ds(i * 16, 16)] = zeros
        return 0
    lax.fori_loop(0, HN // 16, _zero_denom, 0)

    def _p1(g, _):
        off = g * GK
        src_v = src_ref[pl.ds(off, 16)]
        dst_v = dst_ref[pl.ds(off, 16)]
        for h in range(H):
            av = plsc.load_gather(asrc_ref, [src_v + h * N])
            bv = plsc.load_gather(adst_ref, [dst_v + h * N])
            sv = av + bv
            alpha = jnp.where(sv >= 0.0, sv, sv * jnp.float32(0.2))
            ex = jnp.exp(alpha - cmax_ref[...][h])
            expa_ref[pl.ds(h * EPT + off, 16)] = ex
            plsc.addupdate_scatter(denom_ref, [dst_v + h * N], ex)
        return 0
    lax.fori_loop(0, NG, _p1, 0)

    pltpu.sync_copy(denom_ref, shared_den.at[sid])
    plsc.subcore_barrier()

    def _zero_denom2(i, _):
        denom_ref[pl.ds(i * 16, 16)] = zeros
        return 0
    lax.fori_loop(0, HN // 16, _zero_denom2, 0)

    def _accum_tile(t, _):
        pltpu.sync_copy(shared_den.at[t], tmp_ref)

        def _acc(i, _2):
            sl = pl.ds(i * 16, 16)
            denom_ref[sl] = denom_ref[sl] + tmp_ref[sl]
            return 0
        lax.fori_loop(0, HN // 16, _acc, 0)
        return 0
    lax.fori_loop(0, NT, _accum_tile, 0)

    def _csweep(g, _):
        off = g * GK
        dst_v = dst_ref[pl.ds(off, 16)]
        for h in range(H):
            sl = pl.ds(h * EPT + off, 16)
            den = plsc.load_gather(denom_ref, [dst_v + h * N])
            expa_ref[sl] = expa_ref[sl] / (den + jnp.float32(1e-16))
        return 0
    lax.fori_loop(0, NG, _csweep, 0)

    for h in range(H):
        pltpu.sync_copy(expa_ref.at[pl.ds(h * EPT, EPT)],
                        coef_hbm.at[pl.ds(h * E2 + sid * EPT, EPT)])


NW = 2 * NT
RPW = N // NW
CHK = 2048
NCHK = E2 // CHK


def _sc_agg_body(src_hbm, dst_hbm, coef_hbm, hrows_hbm,
                 out_hbm,
                 srcc_ref, dstc_ref, coefc_ref, rows0_ref, rows1_ref,
                 idx0_ref, idx1_ref, pd0_ref, pd1_ref, pc0_ref, pc1_ref,
                 ps_ref, pd_ref, pc_ref, acc_ref, sem0, sem1, semc):
    cid = lax.axis_index("c")
    sid = lax.axis_index("s")
    w = cid * NT + sid
    base = w * RPW
    zeros = jnp.zeros((16,), jnp.float32)
    slots = ((rows0_ref, idx0_ref, pd0_ref, pc0_ref, sem0),
             (rows1_ref, idx1_ref, pd1_ref, pc1_ref, sem1))

    def _start(slot, h):
        rows, idx, pds, pcs, sem = slots[slot]
        idx[...] = ps_ref[pl.ds(0, 16)] + h * N
        pds[...] = pd_ref[pl.ds(0, 16)]
        pcs[...] = pc_ref[pl.ds(0, 16)]
        pltpu.async_copy(hrows_hbm.at[idx], rows, sem)

    def _acc(slot):
        rows, idx, pds, pcs, sem = slots[slot]
        pltpu.make_async_copy(hrows_hbm.at[idx], rows, sem).wait()

        def _row(k, _):
            dl = pds[pl.ds(k, 16)][0]
            ck = pcs[pl.ds(k, 16)][0]
            ab = dl * D
            for j in range(D // 16):
                sl = pl.ds(ab + j * 16, 16)
                acc_ref[sl] = (acc_ref[sl]
                               + rows[k, pl.ds(j * 16, 16)] * ck)
            return 0
        lax.fori_loop(0, GK, _row, 0)

    def _fire(par, nout, h):
        @pl.when((nout == 2) & (par == 0))
        def _():
            _acc(0)

        @pl.when((nout == 2) & (par == 1))
        def _():
            _acc(1)

        @pl.when(par == 0)
        def _():
            _start(0, h)

        @pl.when(par == 1)
        def _():
            _start(1, h)

    def _head(h, _hc):
        def _zero(i, _):
            for j in range(8):
                acc_ref[pl.ds(i * 128 + j * 16, 16)] = zeros
            return 0
        lax.fori_loop(0, RPW * D // 128, _zero, 0)

        def _chunk(cki, carry):
            c1 = pltpu.async_copy(src_hbm.at[pl.ds(cki * CHK, CHK)],
                                  srcc_ref, semc)
            c2 = pltpu.async_copy(dst_hbm.at[pl.ds(cki * CHK, CHK)],
                                  dstc_ref, semc)
            c3 = pltpu.async_copy(coef_hbm.at[pl.ds(h * E2 + cki * CHK, CHK)],
                                  coefc_ref, semc)
            c1.wait()
            c2.wait()
            c3.wait()

            def _grp(i, carry2):
                carry3 = carry2
                for u in range(2):
                    np2, par, nout = carry3
                    off = i * 32 + u * 16
                    dst_v = dstc_ref[pl.ds(off, 16)]
                    dloc = dst_v - base
                    inr = (dloc >= 0) & (dloc < RPW)
                    cnt = plsc.all_reduce_population_count(inr)[0]
                    plsc.store_compressed(ps_ref.at[pl.ds(np2, 16)],
                                          srcc_ref[pl.ds(off, 16)], mask=inr)
                    plsc.store_compressed(pd_ref.at[pl.ds(np2, 16)], dloc,
                                          mask=inr)
                    plsc.store_compressed(pc_ref.at[pl.ds(np2, 16)],
                                          coefc_ref[pl.ds(off, 16)], mask=inr)
                    np3 = np2 + cnt
                    fired = np3 >= GK

                    @pl.when(fired)
                    def _():
                        _fire(par, nout, h)
                        ps_ref[pl.ds(0, 16)] = ps_ref[pl.ds(16, 16)]
                        pd_ref[pl.ds(0, 16)] = pd_ref[pl.ds(16, 16)]
                        pc_ref[pl.ds(0, 16)] = pc_ref[pl.ds(16, 16)]
                    carry3 = (jnp.where(fired, np3 - GK, np3),
                              jnp.where(fired, 1 - par, par),
                              jnp.where(fired,
                                        jnp.minimum(nout + 1, 2), nout))
                return carry3
            return lax.fori_loop(0, CHK // 32, _grp, carry)
        np_f, par_f, nout_f = lax.fori_loop(
            0, NCHK, _chunk,
            (jnp.int32(0), jnp.int32(0), jnp.int32(0)))

        @pl.when(np_f > 0)
        def _():
            lanes = lax.iota(jnp.int32, 16)
            valid = lanes < np_f
            pc_ref[pl.ds(0, 16)] = jnp.where(valid, pc_ref[pl.ds(0, 16)],
                                             jnp.float32(0.0))
            ps_ref[pl.ds(0, 16)] = jnp.where(valid, ps_ref[pl.ds(0, 16)], 0)
            pd_ref[pl.ds(0, 16)] = jnp.where(valid, pd_ref[pl.ds(0, 16)], 0)
            _fire(par_f, nout_f, h)

        fired_t = np_f > 0
        par_t = jnp.where(fired_t, 1 - par_f, par_f)
        nout_t = jnp.where(fired_t, jnp.minimum(nout_f + 1, 2), nout_f)

        @pl.when((nout_t == 2) & (par_t == 0))
        def _():
            _acc(0)

        @pl.when((nout_t == 2) & (par_t == 1))
        def _():
            _acc(1)

        @pl.when((nout_t >= 1) & (par_t == 0))
        def _():
            _acc(1)

        @pl.when((nout_t >= 1) & (par_t == 1))
        def _():
            _acc(0)

        pltpu.sync_copy(acc_ref,
                        out_hbm.at[pl.ds((h * N + base) * D, RPW * D)])
        return 0
    lax.fori_loop(0, H, _head, 0)


def _sc_edge_call(src, dst, asrc_flat, adst_flat, cmax16, hrows):
    mesh = plsc.VectorSubcoreMesh(core_axis_name="c", subcore_axis_name="s")
    coef = functools.partial(
        pl.kernel,
        out_type=jax.ShapeDtypeStruct((H * E2,), jnp.float32),
        mesh=mesh,
        compiler_params=pltpu.CompilerParams(needs_layout_passes=False),
        scratch_types=[
            pltpu.VMEM((EPT,), jnp.int32),
            pltpu.VMEM((EPT,), jnp.int32),
            pltpu.VMEM((HN,), jnp.float32),
            pltpu.VMEM((HN,), jnp.float32),
            pltpu.VMEM((HN,), jnp.float32),
            pltpu.VMEM((HN,), jnp.float32),
            pltpu.VMEM((H * EPT,), jnp.float32),
            pltpu.VMEM((16,), jnp.float32),
            pltpu.VMEM_SHARED((NT, HN), jnp.float32),
        ],
    )(_sc_coef_body)(src, dst, asrc_flat, adst_flat, cmax16)

    agg = functools.partial(
        pl.kernel,
        out_type=jax.ShapeDtypeStruct((HN * D,), jnp.float32),
        mesh=mesh,
        compiler_params=pltpu.CompilerParams(needs_layout_passes=False),
        scratch_types=[
            pltpu.VMEM((CHK,), jnp.int32),
            pltpu.VMEM((CHK,), jnp.int32),
            pltpu.VMEM((CHK,), jnp.float32),
            pltpu.VMEM((GK, D), jnp.float32),
            pltpu.VMEM((GK, D), jnp.float32),
            pltpu.VMEM((GK,), jnp.int32),
            pltpu.VMEM((GK,), jnp.int32),
            pltpu.VMEM((GK,), jnp.int32),
            pltpu.VMEM((GK,), jnp.int32),
            pltpu.VMEM((GK,), jnp.float32),
            pltpu.VMEM((GK,), jnp.float32),
            pltpu.VMEM((2 * GK,), jnp.int32),
            pltpu.VMEM((2 * GK,), jnp.int32),
            pltpu.VMEM((2 * GK,), jnp.float32),
            pltpu.VMEM((RPW * D,), jnp.float32),
            pltpu.SemaphoreType.DMA,
            pltpu.SemaphoreType.DMA,
            pltpu.SemaphoreType.DMA,
        ],
    )(_sc_agg_body)(src, dst, coef, hrows)
    return agg


def kernel(x, edge_index, edge_attr, c, node_batch, Wq, bq, Wk, bk, Wv, bv,
           Wo, bo, W_gat, att_src, att_dst, b_gat):
    del edge_attr
    x3 = x.reshape(N, L, D).astype(jnp.bfloat16)
    nb2 = node_batch.reshape(N, 1)
    wq3 = Wq.reshape(H, DH, D).astype(jnp.bfloat16)
    wk3 = Wk.reshape(H, DH, D).astype(jnp.bfloat16)
    wv3 = Wv.reshape(H, DH, D).astype(jnp.bfloat16)
    woT3 = Wo.T.reshape(H, DH, D).astype(jnp.bfloat16)
    bq2 = bq.reshape(H, DH)
    bk2 = bk.reshape(H, DH)
    bv2 = bv.reshape(H, DH)
    bo2 = bo.reshape(1, D)

    c16 = c.astype(jnp.bfloat16)
    att_x = _mha_call(x3, nb2, c16, wq3, bq2, wk3, bk2, wv3, bv2, woT3, bo2)
    att_x2 = att_x.reshape(N, L * D)

    asrc2 = att_src.reshape(H, 1, D)
    adst2 = att_dst.reshape(H, 1, D)
    h_out, a_srcT, a_dstT = _gat_gemm_call(
        att_x2, W_gat.astype(jnp.bfloat16), asrc2, adst2)

    a_src_flat = a_srcT.reshape(H * N)
    a_dst_flat = a_dstT.reshape(H * N)
    cmax = jax.nn.relu(jnp.max(a_srcT.reshape(H, N), axis=1) +
                       jnp.max(a_dstT.reshape(H, N), axis=1))
    cmax16 = jnp.concatenate([cmax, jnp.zeros((16 - H,), jnp.float32)])
    loop = jnp.arange(N, dtype=edge_index.dtype)
    src = jnp.concatenate([edge_index[0], loop])
    dst = jnp.concatenate([edge_index[1], loop])
    hrows = h_out.reshape(H * N, D)
    agg = _sc_edge_call(src, dst, a_src_flat, a_dst_flat, cmax16, hrows)
    out = agg.reshape(H, N, D).transpose(1, 0, 2).reshape(N, H * D)
    return out + b_gat
